# baseline (speedup 1.0000x reference)
import numpy as np

import jax
import jax.numpy as jnp
from jax import lax
from jax.experimental import pallas as pl
from jax.experimental.pallas import tpu as pltpu
from jax.experimental.pallas import tpu_sc as plsc

B = 4096
N_ENTITY = 100000
N_RELATION = 64
DIM = 32
K = 16

NC = 2
NS = 16
NW = NC * NS
BH = B // 2
NB = BH // NW

CB = 4
H1C = CB * K
H2C = CB * K * K
NCHUNK = NB // CB


def _sc_gather_body(users_hbm, items_hbm, adj_hbm, emb_hbm,
                    u_out, e0_out, e1_out, e2_out, rel0_out, rel1_out,
                    us_v, it_v, urow_v, e0row_v, adj1_v,
                    ent1f_v, adjc_v, ent2f_v, e1b_v, e2c_v,
                    sem_g, sem_a, sem_wr, sem_we, sem_w0):
    wid = lax.axis_index("s") * NC + lax.axis_index("c")
    base = wid * NB

    pltpu.sync_copy(users_hbm.at[pl.ds(base, NB)], us_v)
    pltpu.sync_copy(items_hbm.at[pl.ds(base, NB)], it_v)

    h_u = pltpu.async_copy(emb_hbm.at[us_v], urow_v, sem_g)
    h_e0 = pltpu.async_copy(emb_hbm.at[it_v], e0row_v, sem_g)
    h_a1 = pltpu.async_copy(adj_hbm.at[it_v], adj1_v, sem_a)
    h_u.wait()
    w_u = pltpu.async_copy(urow_v, u_out.at[pl.ds(base, NB)], sem_w0)
    h_e0.wait()
    w_e0 = pltpu.async_copy(e0row_v, e0_out.at[pl.ds(base, NB)], sem_w0)
    h_a1.wait()
    w_r0 = pltpu.async_copy(adj1_v.at[:, pl.ds(K, K)],
                            rel0_out.at[pl.ds(base, NB)], sem_w0)

    def fl1(i, _):
        ent1f_v[pl.ds(i * K, K)] = adj1_v[i, pl.ds(0, K)]
        return 0
    lax.fori_loop(0, NB, fl1, 0, unroll=4)

    for h in range(max(1, (NB * K) // H2C)):
        hs = [pltpu.async_copy(
            emb_hbm.at[ent1f_v.at[pl.ds(h * H2C + s * 128, 128)]],
            e1b_v.at[pl.ds(s * 128, 128)], sem_g) for s in range(8)]
        for hh in hs:
            hh.wait()
        e1w_last = pltpu.async_copy(
            e1b_v, e1_out.at[pl.ds(base * K + h * H2C, H2C)], sem_w0)
        if (h + 1) * H2C < NB * K:
            e1w_last.wait()

    adj_h = {}
    g_h = {}
    wr_h = {}
    we_h = {}

    def fire_adj(c):
        p = c % 2
        idx = ent1f_v.at[pl.ds(c * H1C, H1C)]
        adj_h[c] = pltpu.async_copy(adj_hbm.at[idx], adjc_v.at[p], sem_a)

    QC = H2C // 4
    QW = NB * K * K // 4

    def fire_e2_write(c):
        p = c % 2
        return [pltpu.async_copy(
            e2c_v.at[p, pl.ds(i * QC, QC)],
            e2_out.at[pl.ds(base * K * K + i * QW + c * QC, QC)], sem_we)
            for i in range(4)]

    fire_adj(0)
    for c in range(NCHUNK):
        p = c % 2
        adj_h[c].wait()
        wr_h[c] = pltpu.async_copy(
            adjc_v.at[p].at[:, pl.ds(K, K)],
            rel1_out.at[pl.ds(base * K + c * H1C, H1C)], sem_wr)

        def fl2(i, _, p=p):
            for qtr in range(4):
                ent2f_v[p, pl.ds(qtr * (4 * H1C) + i * 4, 4)] = \
                    adjc_v[p, i, pl.ds(qtr * 4, 4)]
            return 0
        lax.fori_loop(0, H1C, fl2, 0, unroll=4)

        if c >= 2:
            for hh in we_h[c - 2]:
                hh.wait()
        g_h[c] = [pltpu.async_copy(
            emb_hbm.at[ent2f_v.at[p, pl.ds(s * 128, 128)]],
            e2c_v.at[p, pl.ds(s * 128, 128)], sem_g) for s in range(8)]
        if c + 1 < NCHUNK:
            if c >= 1:
                wr_h[c - 1].wait()
            fire_adj(c + 1)
        if c >= 1:
            for hh in g_h[c - 1]:
                hh.wait()
            we_h[c - 1] = fire_e2_write(c - 1)

    c_last = NCHUNK - 1
    for hh in g_h[c_last]:
        hh.wait()
    we_h[c_last] = fire_e2_write(c_last)
    for hh in we_h[c_last - 1]:
        hh.wait()
    for hh in we_h[c_last]:
        hh.wait()
    wr_h[c_last - 1].wait()
    wr_h[c_last].wait()
    w_u.wait()
    w_e0.wait()
    w_r0.wait()
    e1w_last.wait()


@jax.jit
def _sc_gather(users, items, adj, emb):
    mesh = plsc.VectorSubcoreMesh(core_axis_name="c", subcore_axis_name="s",
                                  num_cores=NC, num_subcores=NS)
    f32 = jnp.float32
    i32 = jnp.int32
    out_type = (
        jax.ShapeDtypeStruct((BH, DIM), f32),
        jax.ShapeDtypeStruct((BH, DIM), f32),
        jax.ShapeDtypeStruct((BH * K, DIM), f32),
        jax.ShapeDtypeStruct((BH * K * K, DIM), f32),
        jax.ShapeDtypeStruct((BH, K), i32),
        jax.ShapeDtypeStruct((BH * K, K), i32),
    )
    scratch = [
        pltpu.VMEM((NB,), i32),
        pltpu.VMEM((NB,), i32),
        pltpu.VMEM((NB, DIM), f32),
        pltpu.VMEM((NB, DIM), f32),
        pltpu.VMEM((NB, 2 * K), i32),
        pltpu.VMEM((NB * K,), i32),
        pltpu.VMEM((2, H1C, 2 * K), i32),
        pltpu.VMEM((2, H2C), i32),
        pltpu.VMEM((H2C, DIM), f32),
        pltpu.VMEM((2, H2C, DIM), f32),
        pltpu.SemaphoreType.DMA,
        pltpu.SemaphoreType.DMA,
        pltpu.SemaphoreType.DMA,
        pltpu.SemaphoreType.DMA,
        pltpu.SemaphoreType.DMA,
    ]
    fn = pl.kernel(_sc_gather_body, out_type=out_type, mesh=mesh,
                   scratch_types=scratch,
                   compiler_params=pltpu.CompilerParams(
                       use_tc_tiling_on_sc=False))
    return fn(users, items, adj, emb)


BB = NB
G1 = BB * K
LN = 128
E2R = G1 * K * DIM // LN
E1R = G1 * DIM // LN
UR = BB * DIM // LN

_M2 = np.equal(np.arange(BB * K)[:, None] // K,
               np.arange(BB)[None, :]).astype(np.float32)
_COLQ = np.kron(np.ones((4, 1)), np.eye(DIM)).astype(np.float32)
_P = np.tile(np.eye(K), (1, BB)).astype(np.float32)
_M = np.kron(np.eye(BB), np.ones((1, K))).astype(np.float32)


def _tc_body(u_ref, e0_ref, e1_ref, e2_ref, rel0_ref, rel1_ref,
             relembt_ref, wt_ref, b_ref, m2_ref,
             colq_ref, pm_ref, mm_ref, out_ref):
    U = u_ref[...]
    Wt = wt_ref[...]
    bb = b_ref[...]
    m2 = m2_ref[...]
    colq = colq_ref[...]
    pm = pm_ref[...]
    mm = mm_ref[...]
    rel0 = rel0_ref[...]
    rel1 = rel1_ref[...]

    st = jnp.dot(U, relembt_ref[...],
                 preferred_element_type=jnp.float32)
    mx = jnp.max(st, axis=1, keepdims=True)
    exst = jnp.exp(st - mx)

    exst_g = jnp.dot(m2, exst, preferred_element_type=jnp.float32)
    ex1 = jnp.take_along_axis(exst_g, rel1, axis=1)
    w1 = ex1 / jnp.sum(ex1, axis=1, keepdims=True)
    w1t4 = jnp.concatenate([w1, w1, w1, w1], axis=0)
    ri = lax.broadcasted_iota(jnp.int32, (E2R, LN), 0)
    ci = lax.broadcasted_iota(jnp.int32, (E2R, LN), 1)
    w128 = jnp.take_along_axis(w1t4, 4 * (ri // G1) + ci // DIM, axis=1)
    q = jnp.dot(e2_ref[...] * w128, colq,
                preferred_element_type=jnp.float32)
    n1 = (q[0:G1] + q[G1:2 * G1]
          + q[2 * G1:3 * G1] + q[3 * G1:4 * G1])
    e1 = e1_ref[...]
    h1 = jax.nn.relu(jnp.dot(e1 + n1, Wt,
                             preferred_element_type=jnp.float32) + bb)

    ex0 = jnp.take_along_axis(exst, rel0, axis=1)
    w0 = ex0 / jnp.sum(ex0, axis=1, keepdims=True)
    w0sel = jnp.dot(w0, pm, preferred_element_type=jnp.float32) * mm
    n0 = jnp.dot(w0sel, e1, preferred_element_type=jnp.float32)
    h0 = jax.nn.relu(jnp.dot(e0_ref[...] + n0, Wt,
                             preferred_element_type=jnp.float32) + bb)

    n0p = jnp.dot(w0sel, h1, preferred_element_type=jnp.float32)
    outv = jnp.tanh(jnp.dot(h0 + n0p, Wt,
                            preferred_element_type=jnp.float32) + bb)
    pred = jax.nn.sigmoid(jnp.sum(U * outv, axis=-1, keepdims=True))
    out_ref[...] = pred


def _tc_specs():
    return [
        pl.BlockSpec((BB, DIM), lambda i: (i, 0)),
        pl.BlockSpec((BB, DIM), lambda i: (i, 0)),
        pl.BlockSpec((G1, DIM), lambda i: (i, 0)),
        pl.BlockSpec((E2R, LN), lambda i: (i, 0)),
        pl.BlockSpec((BB, K), lambda i: (i, 0)),
        pl.BlockSpec((G1, K), lambda i: (i, 0)),
        pl.BlockSpec((DIM, N_RELATION), lambda i: (0, 0)),
        pl.BlockSpec((DIM, DIM), lambda i: (0, 0)),
        pl.BlockSpec((1, DIM), lambda i: (0, 0)),
        pl.BlockSpec((G1, BB), lambda i: (0, 0)),
        pl.BlockSpec((LN, DIM), lambda i: (0, 0)),
        pl.BlockSpec((K, G1), lambda i: (0, 0)),
        pl.BlockSpec((BB, G1), lambda i: (0, 0)),
    ]


@jax.jit
def _tc_compute(u, e0, e1, e2, rel0, rel1, relembt, Wt, b2):
    return pl.pallas_call(
        _tc_body,
        grid=(BH // BB,),
        in_specs=_tc_specs(),
        out_specs=pl.BlockSpec((BB, 1), lambda i: (i, 0)),
        out_shape=jax.ShapeDtypeStruct((BH, 1), jnp.float32),
    )(u, e0, e1,
      e2.reshape(BH * K * K * DIM // LN, LN),
      rel0, rel1,
      relembt, Wt, b2,
      jnp.asarray(_M2), jnp.asarray(_COLQ),
      jnp.asarray(_P), jnp.asarray(_M))


def kernel(pairs, adj_entity_np, adj_relation_np, entity_emb, relation_emb,
           W, b):
    users = pairs[:, 0]
    items = pairs[:, 1]
    adj = jnp.concatenate([adj_entity_np, adj_relation_np], axis=1)
    relembt = relation_emb.T
    Wt = W.T
    b2 = b.reshape(1, DIM)
    preds = []
    gathered = [_sc_gather(users[h * BH:(h + 1) * BH],
                           items[h * BH:(h + 1) * BH], adj, entity_emb)
                for h in range(2)]
    for h in range(2):
        u, e0, e1, e2, rel0, rel1 = gathered[h]
        preds.append(_tc_compute(u, e0, e1, e2, rel0, rel1,
                                 relembt, Wt, b2))
    return jnp.concatenate(preds, axis=0).reshape(B)


# speedup vs baseline: 1.1602x; 1.1602x over previous
import numpy as np

import jax
import jax.numpy as jnp
from jax import lax
from jax.experimental import pallas as pl
from jax.experimental.pallas import tpu as pltpu
from jax.experimental.pallas import tpu_sc as plsc

B = 4096
N_ENTITY = 100000
N_RELATION = 64
DIM = 32
K = 16

NC = 2
NS = 16
NW = NC * NS
BH = B // 2
NB = BH // NW

CB = 4
H1C = CB * K
H2C = CB * K * K
NCHUNK = NB // CB


def _sc_gather_body(users_hbm, items_hbm, adj_hbm, emb_hbm,
                    u_out, e0_out, e1_out, e2_out, rel0_out, rel1_out,
                    us_v, it_v, urow_v, e0row_v, adj1_v,
                    ent1f_v, adjc_v, ent2f_v, e1b_v, e2c_v,
                    sem_g, sem_a, sem_wr, sem_we, sem_w0):
    wid = lax.axis_index("s") * NC + lax.axis_index("c")
    base = wid * NB

    pltpu.sync_copy(users_hbm.at[pl.ds(base, NB)], us_v)
    pltpu.sync_copy(items_hbm.at[pl.ds(base, NB)], it_v)

    h_u = pltpu.async_copy(emb_hbm.at[us_v], urow_v, sem_g)
    h_e0 = pltpu.async_copy(emb_hbm.at[it_v], e0row_v, sem_g)
    h_a1 = pltpu.async_copy(adj_hbm.at[it_v], adj1_v, sem_a)
    h_u.wait()
    w_u = pltpu.async_copy(urow_v, u_out.at[pl.ds(base, NB)], sem_w0)
    h_e0.wait()
    w_e0 = pltpu.async_copy(e0row_v, e0_out.at[pl.ds(base, NB)], sem_w0)
    h_a1.wait()
    w_r0 = pltpu.async_copy(adj1_v.at[:, pl.ds(K, K)],
                            rel0_out.at[pl.ds(base, NB)], sem_w0)

    def fl1(i, _):
        ent1f_v[pl.ds(i * K, K)] = adj1_v[i, pl.ds(0, K)]
        return 0
    lax.fori_loop(0, NB, fl1, 0, unroll=4)

    for h in range(max(1, (NB * K) // H2C)):
        hs = [pltpu.async_copy(
            emb_hbm.at[ent1f_v.at[pl.ds(h * H2C + s * 128, 128)]],
            e1b_v.at[pl.ds(s * 128, 128)], sem_g) for s in range(8)]
        for hh in hs:
            hh.wait()
        e1w_last = pltpu.async_copy(
            e1b_v, e1_out.at[pl.ds(base * K + h * H2C, H2C)], sem_w0)
        if (h + 1) * H2C < NB * K:
            e1w_last.wait()

    adj_h = {}
    g_h = {}
    wr_h = {}
    we_h = {}

    def fire_adj(c):
        p = c % 2
        idx = ent1f_v.at[pl.ds(c * H1C, H1C)]
        adj_h[c] = pltpu.async_copy(adj_hbm.at[idx], adjc_v.at[p], sem_a)

    QC = H2C // 4
    QW = NB * K * K // 4

    def fire_e2_write(c):
        p = c % 2
        return [pltpu.async_copy(
            e2c_v.at[p, pl.ds(i * QC, QC)],
            e2_out.at[pl.ds(base * K * K + i * QW + c * QC, QC)], sem_we)
            for i in range(4)]

    fire_adj(0)
    for c in range(NCHUNK):
        p = c % 2
        adj_h[c].wait()
        wr_h[c] = pltpu.async_copy(
            adjc_v.at[p].at[:, pl.ds(K, K)],
            rel1_out.at[pl.ds(base * K + c * H1C, H1C)], sem_wr)

        def fl2(i, _, p=p):
            for qtr in range(4):
                ent2f_v[p, pl.ds(qtr * (4 * H1C) + i * 4, 4)] = \
                    adjc_v[p, i, pl.ds(qtr * 4, 4)]
            return 0
        lax.fori_loop(0, H1C, fl2, 0, unroll=4)

        if c >= 2:
            for hh in we_h[c - 2]:
                hh.wait()
        g_h[c] = [pltpu.async_copy(
            emb_hbm.at[ent2f_v.at[p, pl.ds(s * 128, 128)]],
            e2c_v.at[p, pl.ds(s * 128, 128)], sem_g) for s in range(8)]
        if c + 1 < NCHUNK:
            if c >= 1:
                wr_h[c - 1].wait()
            fire_adj(c + 1)
        if c >= 1:
            for hh in g_h[c - 1]:
                hh.wait()
            we_h[c - 1] = fire_e2_write(c - 1)

    c_last = NCHUNK - 1
    for hh in g_h[c_last]:
        hh.wait()
    we_h[c_last] = fire_e2_write(c_last)
    for hh in we_h[c_last - 1]:
        hh.wait()
    for hh in we_h[c_last]:
        hh.wait()
    wr_h[c_last - 1].wait()
    wr_h[c_last].wait()
    w_u.wait()
    w_e0.wait()
    w_r0.wait()
    e1w_last.wait()


@jax.jit
def _sc_gather(users, items, adj, emb):
    mesh = plsc.VectorSubcoreMesh(core_axis_name="c", subcore_axis_name="s",
                                  num_cores=NC, num_subcores=NS)
    f32 = jnp.float32
    i32 = jnp.int32
    out_type = (
        jax.ShapeDtypeStruct((BH, DIM), f32),
        jax.ShapeDtypeStruct((BH, DIM), f32),
        jax.ShapeDtypeStruct((BH * K, DIM), f32),
        jax.ShapeDtypeStruct((BH * K * K, DIM), f32),
        jax.ShapeDtypeStruct((BH, K), i32),
        jax.ShapeDtypeStruct((BH * K, K), i32),
    )
    scratch = [
        pltpu.VMEM((NB,), i32),
        pltpu.VMEM((NB,), i32),
        pltpu.VMEM((NB, DIM), f32),
        pltpu.VMEM((NB, DIM), f32),
        pltpu.VMEM((NB, 2 * K), i32),
        pltpu.VMEM((NB * K,), i32),
        pltpu.VMEM((2, H1C, 2 * K), i32),
        pltpu.VMEM((2, H2C), i32),
        pltpu.VMEM((H2C, DIM), f32),
        pltpu.VMEM((2, H2C, DIM), f32),
        pltpu.SemaphoreType.DMA,
        pltpu.SemaphoreType.DMA,
        pltpu.SemaphoreType.DMA,
        pltpu.SemaphoreType.DMA,
        pltpu.SemaphoreType.DMA,
    ]
    fn = pl.kernel(_sc_gather_body, out_type=out_type, mesh=mesh,
                   scratch_types=scratch,
                   compiler_params=pltpu.CompilerParams(
                       use_tc_tiling_on_sc=False))
    return fn(users, items, adj, emb)


BB = NB
G1 = BB * K
LN = 128
E2R = G1 * K * DIM // LN
E1R = G1 * DIM // LN
UR = BB * DIM // LN

_M2 = np.equal(np.arange(BB * K)[:, None] // K,
               np.arange(BB)[None, :]).astype(np.float32)
_COLQ = np.kron(np.ones((4, 1)), np.eye(DIM)).astype(np.float32)
_REP4 = np.kron(np.eye(4), np.ones((1, DIM))).astype(np.float32)
_P = np.tile(np.eye(K), (1, BB)).astype(np.float32)
_M = np.kron(np.eye(BB), np.ones((1, K))).astype(np.float32)


def _tc_body(u_ref, e0_ref, e1_ref, e2_ref, rel0_ref, rel1_ref,
             relembt_ref, wt_ref, b_ref, m2_ref,
             colq_ref, rep4_ref, pm_ref, mm_ref, out_ref):
    U = u_ref[...]
    Wt = wt_ref[...]
    bb = b_ref[...]
    m2 = m2_ref[...]
    colq = colq_ref[...]
    pm = pm_ref[...]
    mm = mm_ref[...]
    rel0 = rel0_ref[...]
    rel1 = rel1_ref[...]

    st = jnp.dot(U, relembt_ref[...],
                 preferred_element_type=jnp.float32)
    mx = jnp.max(st, axis=1, keepdims=True)
    exst = jnp.exp(st - mx)

    exst_g = jnp.dot(m2, exst, preferred_element_type=jnp.float32)
    ex1 = jnp.take_along_axis(exst_g, rel1, axis=1)
    w1 = ex1 / jnp.sum(ex1, axis=1, keepdims=True)
    w1q = jnp.concatenate([w1[:, 0:4], w1[:, 4:8],
                           w1[:, 8:12], w1[:, 12:16]], axis=0)
    w128 = jnp.dot(w1q, rep4_ref[...],
                   preferred_element_type=jnp.float32)
    q = jnp.dot(e2_ref[...] * w128, colq,
                preferred_element_type=jnp.float32)
    n1 = (q[0:G1] + q[G1:2 * G1]
          + q[2 * G1:3 * G1] + q[3 * G1:4 * G1])
    e1 = e1_ref[...]
    h1 = jax.nn.relu(jnp.dot(e1 + n1, Wt,
                             preferred_element_type=jnp.float32) + bb)

    ex0 = jnp.take_along_axis(exst, rel0, axis=1)
    w0 = ex0 / jnp.sum(ex0, axis=1, keepdims=True)
    w0sel = jnp.dot(w0, pm, preferred_element_type=jnp.float32) * mm
    n0 = jnp.dot(w0sel, e1, preferred_element_type=jnp.float32)
    h0 = jax.nn.relu(jnp.dot(e0_ref[...] + n0, Wt,
                             preferred_element_type=jnp.float32) + bb)

    n0p = jnp.dot(w0sel, h1, preferred_element_type=jnp.float32)
    outv = jnp.tanh(jnp.dot(h0 + n0p, Wt,
                            preferred_element_type=jnp.float32) + bb)
    pred = jax.nn.sigmoid(jnp.sum(U * outv, axis=-1, keepdims=True))
    out_ref[...] = pred


def _tc_specs():
    return [
        pl.BlockSpec((BB, DIM), lambda i: (i, 0)),
        pl.BlockSpec((BB, DIM), lambda i: (i, 0)),
        pl.BlockSpec((G1, DIM), lambda i: (i, 0)),
        pl.BlockSpec((E2R, LN), lambda i: (i, 0)),
        pl.BlockSpec((BB, K), lambda i: (i, 0)),
        pl.BlockSpec((G1, K), lambda i: (i, 0)),
        pl.BlockSpec((DIM, N_RELATION), lambda i: (0, 0)),
        pl.BlockSpec((DIM, DIM), lambda i: (0, 0)),
        pl.BlockSpec((1, DIM), lambda i: (0, 0)),
        pl.BlockSpec((G1, BB), lambda i: (0, 0)),
        pl.BlockSpec((LN, DIM), lambda i: (0, 0)),
        pl.BlockSpec((4, LN), lambda i: (0, 0)),
        pl.BlockSpec((K, G1), lambda i: (0, 0)),
        pl.BlockSpec((BB, G1), lambda i: (0, 0)),
    ]


@jax.jit
def _tc_compute(u, e0, e1, e2, rel0, rel1, relembt, Wt, b2):
    return pl.pallas_call(
        _tc_body,
        grid=(BH // BB,),
        in_specs=_tc_specs(),
        out_specs=pl.BlockSpec((BB, 1), lambda i: (i, 0)),
        out_shape=jax.ShapeDtypeStruct((BH, 1), jnp.float32),
    )(u, e0, e1,
      e2.reshape(BH * K * K * DIM // LN, LN),
      rel0, rel1,
      relembt, Wt, b2,
      jnp.asarray(_M2), jnp.asarray(_COLQ), jnp.asarray(_REP4),
      jnp.asarray(_P), jnp.asarray(_M))


def kernel(pairs, adj_entity_np, adj_relation_np, entity_emb, relation_emb,
           W, b):
    users = pairs[:, 0]
    items = pairs[:, 1]
    adj = jnp.concatenate([adj_entity_np, adj_relation_np], axis=1)
    relembt = relation_emb.T
    Wt = W.T
    b2 = b.reshape(1, DIM)
    preds = []
    gathered = [_sc_gather(users[h * BH:(h + 1) * BH],
                           items[h * BH:(h + 1) * BH], adj, entity_emb)
                for h in range(2)]
    for h in range(2):
        u, e0, e1, e2, rel0, rel1 = gathered[h]
        preds.append(_tc_compute(u, e0, e1, e2, rel0, rel1,
                                 relembt, Wt, b2))
    return jnp.concatenate(preds, axis=0).reshape(B)
